# baseline (speedup 1.0000x reference)
import functools

import jax
import jax.numpy as jnp
from jax import lax
from jax.experimental import pallas as pl
from jax.experimental.pallas import tpu as pltpu
from jax.experimental.pallas import tpu_sc as plsc

MODEL_DIM = 128
STEP_DIM = MODEL_DIM // 2
SIGNAL_DIM = MODEL_DIM - STEP_DIM
BATCH = 16384
STEP_PAD = 24

_NC, _NS = 2, 16
_NW = _NC * _NS
_BPW = BATCH // _NW

_TBLOCK = 8388608
_PSPAN = 8387712


def _emb_kernel(step_idx_hbm, sig_idx_hbm, step_flat_hbm, sig_flat_hbm,
                out_hbm, sidx_v, pos_v, obuf_v, sem):
    wid = lax.axis_index("s") * _NC + lax.axis_index("c")
    base = wid * _BPW

    pltpu.sync_copy(sig_idx_hbm.at[pl.ds(base, _BPW)], pos_v)
    pltpu.sync_copy(step_idx_hbm.at[pl.ds(base, _BPW)], sidx_v)

    def _pos_body(k, carry):
        r = pos_v[pl.ds(k * 16, 16)]
        pos_v[pl.ds(k * 16, 16)] = ((r >> 7) << 10) | (r & 127)
        return carry

    lax.fori_loop(0, _BPW // 16, _pos_body, 0)

    copies = []
    for c in range(SIGNAL_DIM):
        src = sig_flat_hbm.at[pl.ds((c // 8) * _TBLOCK + (c % 8) * 128, _PSPAN)]
        copies.append(
            pltpu.async_copy(src.at[pos_v], obuf_v.at[STEP_DIM + c], sem))
    for c in range(STEP_DIM):
        src = step_flat_hbm.at[pl.ds(c * STEP_PAD, STEP_PAD)]
        copies.append(pltpu.async_copy(src.at[sidx_v], obuf_v.at[c], sem))
    for cp in copies:
        cp.wait()

    pltpu.sync_copy(obuf_v, out_hbm.at[:, pl.ds(base, _BPW)])


@jax.jit
def _lookup(step_idx, sig_idx, step_flat, sig_flat):
    k = functools.partial(
        pl.kernel,
        out_type=jax.ShapeDtypeStruct((MODEL_DIM, BATCH), jnp.float32),
        mesh=plsc.VectorSubcoreMesh(core_axis_name="c", subcore_axis_name="s"),
        compiler_params=pltpu.CompilerParams(use_tc_tiling_on_sc=False),
        scratch_types=[
            pltpu.VMEM((_BPW,), jnp.int32),
            pltpu.VMEM((_BPW,), jnp.int32),
            pltpu.VMEM((MODEL_DIM, _BPW), jnp.float32),
            pltpu.SemaphoreType.DMA,
        ],
    )(_emb_kernel)
    return k(step_idx, sig_idx, step_flat, sig_flat)


def kernel(step_levels, signal_levels, step_embedding, signal_embedding):
    step_idx = jnp.asarray(step_levels, dtype=jnp.int32)
    sig_idx = jnp.asarray(signal_levels, dtype=jnp.int32)
    sig_flat = (signal_embedding.T.reshape(8, 8, 8192, 128)
                .transpose(0, 2, 1, 3).reshape(-1))
    step_flat = jnp.pad(step_embedding.T, ((0, 0), (0, STEP_PAD - 20))).reshape(-1)
    out_t = _lookup(step_idx, sig_idx, step_flat, sig_flat)
    return out_t.T.astype(jnp.bfloat16)


# speedup vs baseline: 20.7200x; 20.7200x over previous
import functools

import jax
import jax.numpy as jnp
from jax import lax
from jax.experimental import pallas as pl
from jax.experimental.pallas import tpu as pltpu
from jax.experimental.pallas import tpu_sc as plsc

MODEL_DIM = 128
STEP_DIM = MODEL_DIM // 2
SIGNAL_DIM = MODEL_DIM - STEP_DIM
BATCH = 16384
STEP_VOCAB = 20

_NC, _NS = 2, 16
_NW = _NC * _NS
_BPW = BATCH // _NW

_LSPAN = 524232
_CCHUNK = 4


def _emb_kernel(step_idx_hbm, sig_idx_hbm, step_tab_hbm, sig_lines_hbm,
                out_hbm, ridx_v, sidx_v, posl_v, lane_v, stab_v, lbuf_v,
                obuf_v, sem):
    wid = lax.axis_index("s") * _NC + lax.axis_index("c")
    base = wid * _BPW
    iota16 = lax.iota(jnp.int32, 16)

    pltpu.sync_copy(sig_idx_hbm.at[pl.ds(base, _BPW)], ridx_v)
    pltpu.sync_copy(step_idx_hbm.at[pl.ds(base, _BPW)], sidx_v)
    pltpu.sync_copy(step_tab_hbm, stab_v)

    def _pos_body(k, carry):
        r = ridx_v[pl.ds(k * 16, 16)]
        posl_v[pl.ds(k * 16, 16)] = ((r >> 7) << 6) | ((r >> 4) & 7)
        lane_v[pl.ds(k * 16, 16)] = r & 15
        return carry

    lax.fori_loop(0, _BPW // 16, _pos_body, 0)

    def _step_body(k, carry):
        s = sidx_v[pl.ds(k * 16, 16)]
        for c in range(STEP_DIM):
            g = plsc.load_gather(stab_v, [s, jnp.full((16,), c, jnp.int32)])
            obuf_v[c, pl.ds(k * 16, 16)] = g
        return carry

    lax.fori_loop(0, _BPW // 16, _step_body, 0)

    for chunk in range(SIGNAL_DIM // _CCHUNK):
        copies = []
        for dc in range(_CCHUNK):
            c = chunk * _CCHUNK + dc
            src = sig_lines_hbm.at[
                pl.ds((c // 8) * 524288 + (c % 8) * 8, _LSPAN)]
            copies.append(pltpu.async_copy(
                src.at[posl_v], lbuf_v.at[pl.ds(dc * _BPW, _BPW)], sem))
        for cp in copies:
            cp.wait()

        def _ext_body(k, carry):
            lane = lane_v[pl.ds(k * 16, 16)]
            row0 = k * 16 + iota16
            for dc in range(_CCHUNK):
                g = plsc.load_gather(lbuf_v, [row0 + dc * _BPW, lane])
                obuf_v[STEP_DIM + chunk * _CCHUNK + dc,
                       pl.ds(k * 16, 16)] = g
            return carry

        lax.fori_loop(0, _BPW // 16, _ext_body, 0)

    pltpu.sync_copy(obuf_v, out_hbm.at[:, pl.ds(base, _BPW)])


@jax.jit
def _lookup(step_idx, sig_idx, step_tab, sig_lines):
    k = functools.partial(
        pl.kernel,
        out_type=jax.ShapeDtypeStruct((MODEL_DIM, BATCH), jnp.float32),
        mesh=plsc.VectorSubcoreMesh(core_axis_name="c", subcore_axis_name="s"),
        compiler_params=pltpu.CompilerParams(
            use_tc_tiling_on_sc=False, needs_layout_passes=False),
        scratch_types=[
            pltpu.VMEM((_BPW,), jnp.int32),
            pltpu.VMEM((_BPW,), jnp.int32),
            pltpu.VMEM((_BPW,), jnp.int32),
            pltpu.VMEM((_BPW,), jnp.int32),
            pltpu.VMEM((STEP_VOCAB, STEP_DIM), jnp.float32),
            pltpu.VMEM((_CCHUNK * _BPW, 16), jnp.float32),
            pltpu.VMEM((MODEL_DIM, _BPW), jnp.float32),
            pltpu.SemaphoreType.DMA,
        ],
    )(_emb_kernel)
    return k(step_idx, sig_idx, step_tab, sig_lines)


def kernel(step_levels, signal_levels, step_embedding, signal_embedding):
    step_idx = jnp.asarray(step_levels, dtype=jnp.int32)
    sig_idx = jnp.asarray(signal_levels, dtype=jnp.int32)
    sig_lines = (signal_embedding.T.reshape(8, 8, 8192, 128)
                 .transpose(0, 2, 1, 3).reshape(-1, 16))
    out_t = _lookup(step_idx, sig_idx, step_embedding, sig_lines)
    return out_t.T.astype(jnp.bfloat16)


# speedup vs baseline: 25.9398x; 1.2519x over previous
import functools

import jax
import jax.numpy as jnp
from jax import lax
from jax.experimental import pallas as pl
from jax.experimental.pallas import tpu as pltpu
from jax.experimental.pallas import tpu_sc as plsc

MODEL_DIM = 128
STEP_DIM = MODEL_DIM // 2
SIGNAL_DIM = MODEL_DIM - STEP_DIM
BATCH = 16384
STEP_VOCAB = 20

_NC, _NS = 2, 16
_NW = _NC * _NS
_BPW = BATCH // _NW

_LSPAN = 524232
_CCHUNK = 4
_NROUND = SIGNAL_DIM // _CCHUNK


def _emb_kernel(step_idx_hbm, sig_idx_hbm, step_tab_hbm, sig_lines_hbm,
                out_hbm, ridx_v, sidx_v, posl_v, lane_v, stab_v, lbuf_v,
                sobuf_v, robuf_v, sema, semb, osema, osemb, ssem):
    wid = lax.axis_index("s") * _NC + lax.axis_index("c")
    base = wid * _BPW
    iota16 = lax.iota(jnp.int32, 16)
    gsems = (sema, semb)
    osems = (osema, osemb)

    pltpu.sync_copy(sig_idx_hbm.at[pl.ds(base, _BPW)], ridx_v)
    pltpu.sync_copy(step_idx_hbm.at[pl.ds(base, _BPW)], sidx_v)
    pltpu.sync_copy(step_tab_hbm, stab_v)

    def _pos_body(k, carry):
        r = ridx_v[pl.ds(k * 16, 16)]
        posl_v[pl.ds(k * 16, 16)] = ((r >> 7) << 6) | ((r >> 4) & 7)
        lane_v[pl.ds(k * 16, 16)] = r & 15
        return carry

    lax.fori_loop(0, _BPW // 16, _pos_body, 0)

    def _fire(n):
        buf = n % 2
        cps = []
        for dc in range(_CCHUNK):
            c = n * _CCHUNK + dc
            src = sig_lines_hbm.at[
                pl.ds((c // 8) * 524288 + (c % 8) * 8, _LSPAN)]
            cps.append(pltpu.async_copy(
                src.at[posl_v],
                lbuf_v.at[buf, pl.ds(dc * _BPW, _BPW)], gsems[buf]))
        return cps

    pend = {0: _fire(0), 1: _fire(1)}

    def _step_body(k, carry):
        s = sidx_v[pl.ds(k * 16, 16)]
        for c in range(STEP_DIM):
            g = plsc.load_gather(stab_v, [s, jnp.full((16,), c, jnp.int32)])
            sobuf_v[c, pl.ds(k * 16, 16)] = g
        return carry

    lax.fori_loop(0, _BPW // 16, _step_body, 0)
    step_flush = pltpu.async_copy(
        sobuf_v, out_hbm.at[pl.ds(0, STEP_DIM), pl.ds(base, _BPW)], ssem)

    oflush = {}
    for n in range(_NROUND):
        buf = n % 2
        for cp in pend.pop(n):
            cp.wait()
        if n - 2 in oflush:
            oflush.pop(n - 2).wait()

        def _ext_body(k, carry, buf=buf):
            lane = lane_v[pl.ds(k * 16, 16)]
            row0 = k * 16 + iota16
            for dc in range(_CCHUNK):
                g = plsc.load_gather(
                    lbuf_v.at[buf], [row0 + dc * _BPW, lane])
                robuf_v[buf, dc, pl.ds(k * 16, 16)] = g
            return carry

        lax.fori_loop(0, _BPW // 16, _ext_body, 0)
        if n + 2 < _NROUND:
            pend[n + 2] = _fire(n + 2)
        oflush[n] = pltpu.async_copy(
            robuf_v.at[buf],
            out_hbm.at[pl.ds(STEP_DIM + n * _CCHUNK, _CCHUNK),
                       pl.ds(base, _BPW)], osems[buf])

    for n in sorted(oflush):
        oflush[n].wait()
    step_flush.wait()


@jax.jit
def _lookup(step_idx, sig_idx, step_tab, sig_lines):
    k = functools.partial(
        pl.kernel,
        out_type=jax.ShapeDtypeStruct((MODEL_DIM, BATCH), jnp.float32),
        mesh=plsc.VectorSubcoreMesh(core_axis_name="c", subcore_axis_name="s"),
        compiler_params=pltpu.CompilerParams(
            use_tc_tiling_on_sc=False, needs_layout_passes=False),
        scratch_types=[
            pltpu.VMEM((_BPW,), jnp.int32),
            pltpu.VMEM((_BPW,), jnp.int32),
            pltpu.VMEM((_BPW,), jnp.int32),
            pltpu.VMEM((_BPW,), jnp.int32),
            pltpu.VMEM((STEP_VOCAB, STEP_DIM), jnp.float32),
            pltpu.VMEM((2, _CCHUNK * _BPW, 16), jnp.float32),
            pltpu.VMEM((STEP_DIM, _BPW), jnp.float32),
            pltpu.VMEM((2, _CCHUNK, _BPW), jnp.float32),
            pltpu.SemaphoreType.DMA,
            pltpu.SemaphoreType.DMA,
            pltpu.SemaphoreType.DMA,
            pltpu.SemaphoreType.DMA,
            pltpu.SemaphoreType.DMA,
        ],
    )(_emb_kernel)
    return k(step_idx, sig_idx, step_tab, sig_lines)


def kernel(step_levels, signal_levels, step_embedding, signal_embedding):
    step_idx = jnp.asarray(step_levels, dtype=jnp.int32)
    sig_idx = jnp.asarray(signal_levels, dtype=jnp.int32)
    sig_lines = (signal_embedding.T.reshape(8, 8, 8192, 128)
                 .transpose(0, 2, 1, 3).reshape(-1, 16))
    out_t = _lookup(step_idx, sig_idx, step_embedding, sig_lines)
    return out_t.T.astype(jnp.bfloat16)
